# baseline (speedup 1.0000x reference)
import functools

import jax
import jax.numpy as jnp
from jax import lax
from jax.experimental import pallas as pl
from jax.experimental.pallas import tpu as pltpu
from jax.experimental.pallas import tpu_sc as plsc

B = 4096
L = 200
D = 64
NC, NS = 2, 16
NW = NC * NS
EPT = B // NW
C0, C1 = 96, 104
NBUF = 4


def _linearize(a):
  n = a.size
  assert n % 1024 == 0
  a3 = a.reshape(n // 1024, 8, 128)
  a3 = lax.optimization_barrier(a3)
  return a3.reshape(a.shape)


def _xpose_body(x_ref, o_ref):
  eye = jnp.float32(1.0) * (lax.broadcasted_iota(jnp.int32, (64, 64), 0) ==
                            lax.broadcasted_iota(jnp.int32, (64, 64), 1))
  y = lax.dot_general(x_ref[...], eye, (((0,), (0,)), ((), ())),
                      preferred_element_type=jnp.float32)
  y3 = y.reshape(1024, 2, 64)
  o_ref[...] = jnp.concatenate([y3[:, 0, :], y3[:, 1, :]], axis=-1)


def _xpose(a):
  v = a.shape[0]
  grid = (v + 2047) // 2048
  out = pl.pallas_call(
      _xpose_body,
      grid=(grid,),
      in_specs=[pl.BlockSpec((64, 2048), lambda i: (0, i))],
      out_specs=pl.BlockSpec((1024, 128), lambda i: (i, 0)),
      out_shape=jax.ShapeDtypeStruct((v // 2, 128), jnp.float32),
  )(a.T)
  return out.reshape(v, 64)


def _sc_pool_body(x_h, table_h, outh, idx_v, rows_v, out_v, sems):
  cid = lax.axis_index("c")
  sid = lax.axis_index("s")
  wid = sid * NC + cid
  base = wid * EPT

  pltpu.sync_copy(x_h.at[pl.ds(base, EPT)], idx_v)

  def issue(e, b):
    pltpu.async_copy(table_h.at[idx_v.at[e, pl.ds(0, C0)]],
                     rows_v.at[b, pl.ds(0, C0)], sems.at[b])
    pltpu.async_copy(table_h.at[idx_v.at[e, pl.ds(C0, C1)]],
                     rows_v.at[b, pl.ds(C0, C1)], sems.at[b])

  def wait(b):
    pltpu.make_async_copy(table_h.at[pl.ds(0, L)], rows_v.at[b],
                          sems.at[b]).wait()

  def accum_store(e, b):
    def row_add(r, acc):
      return (acc[0] + rows_v[b, r, pl.ds(0, 16)],
              acc[1] + rows_v[b, r, pl.ds(16, 16)],
              acc[2] + rows_v[b, r, pl.ds(32, 16)],
              acc[3] + rows_v[b, r, pl.ds(48, 16)])

    z = jnp.zeros((16,), jnp.float32)
    a0, a1, a2, a3 = lax.fori_loop(0, L, row_add, (z, z, z, z), unroll=8)
    out_v[e, pl.ds(0, 16)] = a0
    out_v[e, pl.ds(16, 16)] = a1
    out_v[e, pl.ds(32, 16)] = a2
    out_v[e, pl.ds(48, 16)] = a3

  for b in range(NBUF):
    issue(b, b)

  def group(k, carry):
    for b in range(NBUF):
      e = k * NBUF + b
      wait(b)
      accum_store(e, b)

      @pl.when(k < EPT // NBUF - 1)
      def _(e=e, b=b):
        issue(e + NBUF, b)
    return carry

  lax.fori_loop(0, EPT // NBUF, group, 0)

  pltpu.sync_copy(out_v, outh.at[pl.ds(base, EPT)])


_sc_pool = functools.partial(
    pl.kernel,
    out_type=jax.ShapeDtypeStruct((B, D), jnp.float32),
    mesh=plsc.VectorSubcoreMesh(core_axis_name="c", subcore_axis_name="s"),
    compiler_params=pltpu.CompilerParams(use_tc_tiling_on_sc=False),
    scratch_types=[
        pltpu.VMEM((EPT, L), jnp.int32),
        pltpu.VMEM((NBUF, L, D), jnp.float32),
        pltpu.VMEM((EPT, D), jnp.float32),
        pltpu.SemaphoreType.DMA((NBUF,)),
    ],
)(_sc_pool_body)


def _mlp_body(p0_ref, p1_ref, p2_ref, w1_ref, b1_ref, w2_ref, b2_ref, o_ref):
  s = 1.0 / L
  h = lax.dot_general(p0_ref[...] * s, w1_ref[:, 0:D], (((1,), (1,)), ((), ())),
                      preferred_element_type=jnp.float32)
  h += lax.dot_general(p1_ref[...] * s, w1_ref[:, D:2 * D],
                       (((1,), (1,)), ((), ())),
                       preferred_element_type=jnp.float32)
  h += lax.dot_general(p2_ref[...] * s, w1_ref[:, 2 * D:3 * D],
                       (((1,), (1,)), ((), ())),
                       preferred_element_type=jnp.float32)
  h = jnp.maximum(h + b1_ref[...], 0.0)
  o = lax.dot_general(h, w2_ref[...], (((1,), (1,)), ((), ())),
                      preferred_element_type=jnp.float32)
  o_ref[...] = o + b2_ref[...]


def _mlp(p0, p1, p2, fc1_w, fc1_b, fc2_w, fc2_b):
  H = fc1_w.shape[0]
  C = fc2_w.shape[0]
  tb = 512
  return pl.pallas_call(
      _mlp_body,
      grid=(B // tb,),
      in_specs=[
          pl.BlockSpec((tb, D), lambda i: (i, 0)),
          pl.BlockSpec((tb, D), lambda i: (i, 0)),
          pl.BlockSpec((tb, D), lambda i: (i, 0)),
          pl.BlockSpec((H, 3 * D), lambda i: (0, 0)),
          pl.BlockSpec((1, H), lambda i: (0, 0)),
          pl.BlockSpec((C, H), lambda i: (0, 0)),
          pl.BlockSpec((1, C), lambda i: (0, 0)),
      ],
      out_specs=pl.BlockSpec((tb, C), lambda i: (i, 0)),
      out_shape=jax.ShapeDtypeStruct((B, C), jnp.float32),
  )(p0, p1, p2, fc1_w, fc1_b.reshape(1, H), fc2_w, fc2_b.reshape(1, C))


@jax.jit
def kernel(x0, x1, x2, x3, W_word, W_bi, W_tri, fc1_w, fc1_b, fc2_w, fc2_b):
  del x1
  x0l = _linearize(x0.astype(jnp.int32))
  x2l = _linearize(x2.astype(jnp.int32))
  x3l = _linearize(x3.astype(jnp.int32))
  p0 = _sc_pool(x0l, _xpose(W_word))
  p1 = _sc_pool(x2l, _xpose(W_bi))
  p2 = _sc_pool(x3l, _xpose(W_tri))
  return _mlp(p0, p1, p2, fc1_w, fc1_b, fc2_w, fc2_b)


# speedup vs baseline: 1.3899x; 1.3899x over previous
import functools

import jax
import jax.numpy as jnp
from jax import lax
from jax.experimental import pallas as pl
from jax.experimental.pallas import tpu as pltpu
from jax.experimental.pallas import tpu_sc as plsc

B = 4096
L = 200
D = 64
NC, NS = 2, 16
NW = NC * NS
EPT = B // NW
C0, C1 = 96, 104
LP = 208
NBUF = 4


def _linearize(a):
  n = a.size
  assert n % 1024 == 0
  a3 = a.reshape(n // 1024, 8, 128)
  a3 = lax.optimization_barrier(a3)
  return a3.reshape(a.shape)


def _xpose_body(x_ref, o_ref):
  x = x_ref[...]
  z = jnp.concatenate([x[:, 0:1024], x[:, 1024:2048]], axis=0)
  o_ref[...] = z.T


def _xpose(a):
  v = a.shape[0]
  grid = (v + 2047) // 2048
  out = pl.pallas_call(
      _xpose_body,
      grid=(grid,),
      in_specs=[pl.BlockSpec((64, 2048), lambda i: (0, i))],
      out_specs=pl.BlockSpec((1024, 128), lambda i: (i, 0)),
      out_shape=jax.ShapeDtypeStruct((grid * 1024, 128), jnp.float32),
  )(a.T)
  return out.reshape(grid * 2048, D)


def _sc_pool_body(x_h, table_h, outh, idx_v, rows_v, out_v, sems):
  cid = lax.axis_index("c")
  sid = lax.axis_index("s")
  wid = sid * NC + cid
  base = wid * EPT

  pltpu.sync_copy(x_h.at[pl.ds(base, EPT)], idx_v.at[:, pl.ds(0, L)])

  def remap(e):
    for k in range(LP // 16):
      r = idx_v[e, pl.ds(16 * k, 16)]
      q = (((r >> 11) << 11) | ((r & 1023) << 1) | ((r >> 10) & 1))
      idx_v[e, pl.ds(16 * k, 16)] = q

  def issue(e, b):
    pltpu.async_copy(table_h.at[idx_v.at[e, pl.ds(0, C0)]],
                     rows_v.at[b, pl.ds(0, C0)], sems.at[b])
    pltpu.async_copy(table_h.at[idx_v.at[e, pl.ds(C0, C1)]],
                     rows_v.at[b, pl.ds(C0, C1)], sems.at[b])

  def wait(b):
    pltpu.make_async_copy(table_h.at[pl.ds(0, L)], rows_v.at[b],
                          sems.at[b]).wait()

  def accum_store(e, b):
    def row_add(r, acc):
      return (acc[0] + rows_v[b, r, pl.ds(0, 16)],
              acc[1] + rows_v[b, r, pl.ds(16, 16)],
              acc[2] + rows_v[b, r, pl.ds(32, 16)],
              acc[3] + rows_v[b, r, pl.ds(48, 16)])

    z = jnp.zeros((16,), jnp.float32)
    a0, a1, a2, a3 = lax.fori_loop(0, L, row_add, (z, z, z, z), unroll=8)
    out_v[e, pl.ds(0, 16)] = a0
    out_v[e, pl.ds(16, 16)] = a1
    out_v[e, pl.ds(32, 16)] = a2
    out_v[e, pl.ds(48, 16)] = a3

  for b in range(NBUF):
    remap(b)
    issue(b, b)

  def group(k, carry):
    for b in range(NBUF):
      e = k * NBUF + b
      wait(b)
      accum_store(e, b)

      @pl.when(k < EPT // NBUF - 1)
      def _(e=e, b=b):
        remap(e + NBUF)
        issue(e + NBUF, b)
    return carry

  lax.fori_loop(0, EPT // NBUF, group, 0)

  pltpu.sync_copy(out_v, outh.at[pl.ds(base, EPT)])


_sc_pool = functools.partial(
    pl.kernel,
    out_type=jax.ShapeDtypeStruct((B, D), jnp.float32),
    mesh=plsc.VectorSubcoreMesh(core_axis_name="c", subcore_axis_name="s"),
    compiler_params=pltpu.CompilerParams(use_tc_tiling_on_sc=False),
    scratch_types=[
        pltpu.VMEM((EPT, LP), jnp.int32),
        pltpu.VMEM((NBUF, L, D), jnp.float32),
        pltpu.VMEM((EPT, D), jnp.float32),
        pltpu.SemaphoreType.DMA((NBUF,)),
    ],
)(_sc_pool_body)


def _mlp_body(p0_ref, p1_ref, p2_ref, w1_ref, b1_ref, w2_ref, b2_ref, o_ref):
  s = 1.0 / L
  h = lax.dot_general(p0_ref[...] * s, w1_ref[:, 0:D], (((1,), (1,)), ((), ())),
                      preferred_element_type=jnp.float32)
  h += lax.dot_general(p1_ref[...] * s, w1_ref[:, D:2 * D],
                       (((1,), (1,)), ((), ())),
                       preferred_element_type=jnp.float32)
  h += lax.dot_general(p2_ref[...] * s, w1_ref[:, 2 * D:3 * D],
                       (((1,), (1,)), ((), ())),
                       preferred_element_type=jnp.float32)
  h = jnp.maximum(h + b1_ref[...], 0.0)
  o = lax.dot_general(h, w2_ref[...], (((1,), (1,)), ((), ())),
                      preferred_element_type=jnp.float32)
  o_ref[...] = o + b2_ref[...]


def _mlp(p0, p1, p2, fc1_w, fc1_b, fc2_w, fc2_b):
  H = fc1_w.shape[0]
  C = fc2_w.shape[0]
  tb = 512
  return pl.pallas_call(
      _mlp_body,
      grid=(B // tb,),
      in_specs=[
          pl.BlockSpec((tb, D), lambda i: (i, 0)),
          pl.BlockSpec((tb, D), lambda i: (i, 0)),
          pl.BlockSpec((tb, D), lambda i: (i, 0)),
          pl.BlockSpec((H, 3 * D), lambda i: (0, 0)),
          pl.BlockSpec((1, H), lambda i: (0, 0)),
          pl.BlockSpec((C, H), lambda i: (0, 0)),
          pl.BlockSpec((1, C), lambda i: (0, 0)),
      ],
      out_specs=pl.BlockSpec((tb, C), lambda i: (i, 0)),
      out_shape=jax.ShapeDtypeStruct((B, C), jnp.float32),
  )(p0, p1, p2, fc1_w, fc1_b.reshape(1, H), fc2_w, fc2_b.reshape(1, C))


@jax.jit
def kernel(x0, x1, x2, x3, W_word, W_bi, W_tri, fc1_w, fc1_b, fc2_w, fc2_b):
  del x1
  x0l = _linearize(x0.astype(jnp.int32))
  x2l = _linearize(x2.astype(jnp.int32))
  x3l = _linearize(x3.astype(jnp.int32))
  p0 = _sc_pool(x0l, _xpose(W_word))
  p1 = _sc_pool(x2l, _xpose(W_bi))
  p2 = _sc_pool(x3l, _xpose(W_tri))
  return _mlp(p0, p1, p2, fc1_w, fc1_b, fc2_w, fc2_b)


# speedup vs baseline: 2.2287x; 1.6034x over previous
import functools

import jax
import jax.numpy as jnp
from jax import lax
from jax.experimental import pallas as pl
from jax.experimental.pallas import tpu as pltpu
from jax.experimental.pallas import tpu_sc as plsc

B = 4096
L = 200
D = 64
NC, NS = 2, 16
NW = NC * NS
EPT = B // NW
C0, C1 = 96, 104
LP = 208
NBUF = 4
XB = 16384
XH = XB // 2


def _linearize(a):
  n = a.size
  assert n % 1024 == 0
  a3 = a.reshape(n // 1024, 8, 128)
  a3 = lax.optimization_barrier(a3)
  return a3.reshape(a.shape)


def _xpose_body(x_ref, o_ref):
  x = x_ref[...]
  z = jnp.concatenate([x[:, 0:XH], x[:, XH:XB]], axis=0)
  o_ref[...] = z.T


def _xpose(a):
  v = a.shape[0]
  grid = (v + XB - 1) // XB
  out = pl.pallas_call(
      _xpose_body,
      grid=(grid,),
      in_specs=[pl.BlockSpec((64, XB), lambda i: (0, i))],
      out_specs=pl.BlockSpec((XH, 128), lambda i: (i, 0)),
      out_shape=jax.ShapeDtypeStruct((grid * XH, 128), jnp.float32),
  )(a.T)
  return out.reshape(grid * XB, D)


def _sc_pool_body(x_h, table_h, outh, idx_v, rows_v, out_v, sems):
  cid = lax.axis_index("c")
  sid = lax.axis_index("s")
  wid = sid * NC + cid
  base = wid * EPT

  pltpu.sync_copy(x_h.at[pl.ds(base, EPT)], idx_v.at[:, pl.ds(0, L)])

  def remap(e):
    for k in range(LP // 16):
      r = idx_v[e, pl.ds(16 * k, 16)]
      q = (((r >> 14) << 14) | ((r & (XH - 1)) << 1) | ((r >> 13) & 1))
      idx_v[e, pl.ds(16 * k, 16)] = q

  def issue(e, b):
    pltpu.async_copy(table_h.at[idx_v.at[e, pl.ds(0, C0)]],
                     rows_v.at[b, pl.ds(0, C0)], sems.at[b])
    pltpu.async_copy(table_h.at[idx_v.at[e, pl.ds(C0, C1)]],
                     rows_v.at[b, pl.ds(C0, C1)], sems.at[b])

  def wait(b):
    pltpu.make_async_copy(table_h.at[pl.ds(0, L)], rows_v.at[b],
                          sems.at[b]).wait()

  def accum_store(e, b):
    def row_add(r, acc):
      return (acc[0] + rows_v[b, r, pl.ds(0, 16)],
              acc[1] + rows_v[b, r, pl.ds(16, 16)],
              acc[2] + rows_v[b, r, pl.ds(32, 16)],
              acc[3] + rows_v[b, r, pl.ds(48, 16)])

    z = jnp.zeros((16,), jnp.float32)
    a0, a1, a2, a3 = lax.fori_loop(0, L, row_add, (z, z, z, z), unroll=8)
    out_v[e, pl.ds(0, 16)] = a0
    out_v[e, pl.ds(16, 16)] = a1
    out_v[e, pl.ds(32, 16)] = a2
    out_v[e, pl.ds(48, 16)] = a3

  for b in range(NBUF):
    remap(b)
    issue(b, b)

  def group(k, carry):
    for b in range(NBUF):
      e = k * NBUF + b
      wait(b)
      accum_store(e, b)

      @pl.when(k < EPT // NBUF - 1)
      def _(e=e, b=b):
        remap(e + NBUF)
        issue(e + NBUF, b)
    return carry

  lax.fori_loop(0, EPT // NBUF, group, 0)

  pltpu.sync_copy(out_v, outh.at[pl.ds(base, EPT)])


_sc_pool = functools.partial(
    pl.kernel,
    out_type=jax.ShapeDtypeStruct((B, D), jnp.float32),
    mesh=plsc.VectorSubcoreMesh(core_axis_name="c", subcore_axis_name="s"),
    compiler_params=pltpu.CompilerParams(use_tc_tiling_on_sc=False),
    scratch_types=[
        pltpu.VMEM((EPT, LP), jnp.int32),
        pltpu.VMEM((NBUF, L, D), jnp.float32),
        pltpu.VMEM((EPT, D), jnp.float32),
        pltpu.SemaphoreType.DMA((NBUF,)),
    ],
)(_sc_pool_body)


def _mlp_body(p0_ref, p1_ref, p2_ref, w1_ref, b1_ref, w2_ref, b2_ref, o_ref):
  s = 1.0 / L
  h = lax.dot_general(p0_ref[...] * s, w1_ref[:, 0:D], (((1,), (1,)), ((), ())),
                      preferred_element_type=jnp.float32)
  h += lax.dot_general(p1_ref[...] * s, w1_ref[:, D:2 * D],
                       (((1,), (1,)), ((), ())),
                       preferred_element_type=jnp.float32)
  h += lax.dot_general(p2_ref[...] * s, w1_ref[:, 2 * D:3 * D],
                       (((1,), (1,)), ((), ())),
                       preferred_element_type=jnp.float32)
  h = jnp.maximum(h + b1_ref[...], 0.0)
  o = lax.dot_general(h, w2_ref[...], (((1,), (1,)), ((), ())),
                      preferred_element_type=jnp.float32)
  o_ref[...] = o + b2_ref[...]


def _mlp(p0, p1, p2, fc1_w, fc1_b, fc2_w, fc2_b):
  H = fc1_w.shape[0]
  C = fc2_w.shape[0]
  tb = 512
  return pl.pallas_call(
      _mlp_body,
      grid=(B // tb,),
      in_specs=[
          pl.BlockSpec((tb, D), lambda i: (i, 0)),
          pl.BlockSpec((tb, D), lambda i: (i, 0)),
          pl.BlockSpec((tb, D), lambda i: (i, 0)),
          pl.BlockSpec((H, 3 * D), lambda i: (0, 0)),
          pl.BlockSpec((1, H), lambda i: (0, 0)),
          pl.BlockSpec((C, H), lambda i: (0, 0)),
          pl.BlockSpec((1, C), lambda i: (0, 0)),
      ],
      out_specs=pl.BlockSpec((tb, C), lambda i: (i, 0)),
      out_shape=jax.ShapeDtypeStruct((B, C), jnp.float32),
  )(p0, p1, p2, fc1_w, fc1_b.reshape(1, H), fc2_w, fc2_b.reshape(1, C))


@jax.jit
def kernel(x0, x1, x2, x3, W_word, W_bi, W_tri, fc1_w, fc1_b, fc2_w, fc2_b):
  del x1
  x0l = _linearize(x0.astype(jnp.int32))
  x2l = _linearize(x2.astype(jnp.int32))
  x3l = _linearize(x3.astype(jnp.int32))
  p0 = _sc_pool(x0l, _xpose(W_word))
  p1 = _sc_pool(x2l, _xpose(W_bi))
  p2 = _sc_pool(x3l, _xpose(W_tri))
  return _mlp(p0, p1, p2, fc1_w, fc1_b, fc2_w, fc2_b)
